# Initial kernel scaffold; baseline (speedup 1.0000x reference)
import jax
import jax.numpy as jnp
from jax.experimental import pallas as pl


def kernel(inputs, embed_weight):
    raise NotImplementedError("write your pallas kernel here")


# speedup vs baseline: 5.3351x; 5.3351x over previous
import functools

import jax
import jax.numpy as jnp
from jax import lax
from jax.experimental import pallas as pl
from jax.experimental.pallas import tpu as pltpu
from jax.experimental.pallas import tpu_sc as plsc

ROWS, COLS = 16384, 200
VOCAB, DIM = 6, 4
PACK = 4
GDIM = PACK * DIM
NCOMB = VOCAB ** PACK
CCOLS = COLS // PACK
NG = ROWS * CCOLS

NC, NS = 2, 16
NW = NC * NS
STREAM = 128
CH = 8
G_STEP = CH * STREAM
G_W = NG // NW
STEPS = G_W // G_STEP
C_ROWS = NG // STREAM

_TC_BLK = 2048


def _cidx_body(idx_ref, c_ref):
    idxf = idx_ref[...].astype(jnp.float32)
    l = lax.broadcasted_iota(jnp.int32, (COLS, CCOLS), 0)
    j = lax.broadcasted_iota(jnp.int32, (COLS, CCOLS), 1)
    m = l % PACK
    wgt = jnp.where(m == 0, 216.0, jnp.where(m == 1, 36.0, jnp.where(m == 2, 6.0, 1.0)))
    mat = jnp.where(l // PACK == j, wgt, 0.0)
    c = jax.lax.dot(idxf, mat, preferred_element_type=jnp.float32)
    c_ref[...] = c.astype(jnp.int32)


def _combined_indices(idx):
    return pl.pallas_call(
        _cidx_body,
        out_shape=jax.ShapeDtypeStruct((ROWS, CCOLS), jnp.int32),
        grid=(ROWS // _TC_BLK,),
        in_specs=[pl.BlockSpec((_TC_BLK, COLS), lambda i: (i, 0))],
        out_specs=pl.BlockSpec((_TC_BLK, CCOLS), lambda i: (i, 0)),
    )(idx)


def _product_table(embed_weight):
    ar = jnp.arange(NCOMB, dtype=jnp.int32)
    digs = jnp.stack(
        [ar // 216 % VOCAB, ar // 36 % VOCAB, ar // VOCAB % VOCAB, ar % VOCAB], axis=1
    )
    onehot = (digs[:, :, None] == jnp.arange(VOCAB)).astype(jnp.float32)
    table = jnp.sum(onehot[:, :, :, None] * embed_weight[None, None, :, :], axis=2)
    return table.reshape(NCOMB, GDIM)


def _gather_body(table_hbm, c_hbm, out_hbm, cbuf, rowsbuf, sem):
    wid = lax.axis_index("s") * NC + lax.axis_index("c")
    base_row = wid * (G_W // STREAM)

    def step(g, carry):
        r0 = base_row + g * CH
        pltpu.sync_copy(c_hbm.at[pl.ds(r0, CH)], cbuf)
        copies = [
            pltpu.async_copy(
                table_hbm.at[cbuf.at[j]],
                rowsbuf.at[pl.ds(j * STREAM, STREAM)],
                sem,
            )
            for j in range(CH)
        ]
        for c in copies:
            c.wait()
        pltpu.sync_copy(rowsbuf, out_hbm.at[pl.ds(r0 * STREAM, G_STEP)])
        return carry

    lax.fori_loop(0, STEPS, step, 0)


def kernel(inputs, embed_weight):
    idx = inputs.astype(jnp.int32)
    table = _product_table(embed_weight.astype(jnp.float32))
    c = _combined_indices(idx).reshape(C_ROWS, STREAM)
    mesh = plsc.VectorSubcoreMesh(
        core_axis_name="c", subcore_axis_name="s", num_cores=NC, num_subcores=NS
    )
    out = pl.kernel(
        _gather_body,
        out_type=jax.ShapeDtypeStruct((NG, GDIM), jnp.float32),
        mesh=mesh,
        scratch_types=[
            pltpu.VMEM((CH, STREAM), jnp.int32),
            pltpu.VMEM((G_STEP, GDIM), jnp.float32),
            pltpu.SemaphoreType.DMA,
        ],
        compiler_params=pltpu.CompilerParams(use_tc_tiling_on_sc=False),
    )(table, c)
    return out.reshape(ROWS, COLS, DIM)


# speedup vs baseline: 52.2997x; 9.8029x over previous
import jax
import jax.numpy as jnp
from jax import lax
from jax.experimental import pallas as pl
from jax.experimental.pallas import tpu as pltpu
from jax.experimental.pallas import tpu_sc as plsc

ROWS, COLS = 16384, 200
VOCAB, DIM = 6, 4
PACK = 4
GDIM = PACK * DIM
NCOMB = VOCAB ** PACK
NG = ROWS * COLS // PACK
NOUT = ROWS * COLS * DIM

NC, NS = 2, 16
NW = NC * NS
STREAM = 128
CH = 8
G_STEP = CH * STREAM
G_W = NG // NW
STEPS = G_W // G_STEP
C_ROWS = NG // STREAM

_TCJ = 8
_KC = 512


def _cidx_body(a_ref, c_ref):
    kio = lax.broadcasted_iota(jnp.int32, (_KC, 128), 0)
    mio = lax.broadcasted_iota(jnp.int32, (_KC, 128), 1)
    a = kio % PACK
    wgt = jnp.where(a == 0, 216.0, jnp.where(a == 1, 36.0, jnp.where(a == 2, 6.0, 1.0)))
    m00 = jnp.where(kio // PACK == mio, wgt, 0.0)
    af = a_ref[...].astype(jnp.float32)
    for t in range(ROWS // _KC):
        blk = af[:, _KC * t:_KC * (t + 1)]
        c = jax.lax.dot(blk, m00, preferred_element_type=jnp.float32)
        c_ref[:, t, :] = c.astype(jnp.int32)


def _combined_indices(a):
    return pl.pallas_call(
        _cidx_body,
        out_shape=jax.ShapeDtypeStruct((COLS, ROWS // _KC, 128), jnp.int32),
        grid=(COLS // _TCJ,),
        in_specs=[pl.BlockSpec((_TCJ, ROWS), lambda i: (i, 0))],
        out_specs=pl.BlockSpec((_TCJ, ROWS // _KC, 128), lambda i: (i, 0, 0)),
    )(a)


def _product_table(embed_weight):
    ar = jnp.arange(NCOMB, dtype=jnp.int32)
    digs = jnp.stack(
        [ar // 216 % VOCAB, ar // 36 % VOCAB, ar // VOCAB % VOCAB, ar % VOCAB], axis=1
    )
    onehot = (digs[:, :, None] == jnp.arange(VOCAB)).astype(jnp.float32)
    table = jnp.sum(onehot[:, :, :, None] * embed_weight[None, None, :, :], axis=2)
    return table.reshape(NCOMB, GDIM)


def _gather_body(table_hbm, c_hbm, out_hbm, cbuf, rowsbuf, obuf, sem):
    wid = lax.axis_index("s") * NC + lax.axis_index("c")
    base_crow = wid * (G_W // STREAM)
    iot = lax.iota(jnp.int32, 16)
    colv = [(d + 4 * iot) & 15 for d in range(DIM)]
    rowv = [(d + 4 * iot) >> 4 for d in range(DIM)]

    def step(s, carry):
        r0 = base_crow + s * CH
        pltpu.sync_copy(c_hbm.at[pl.ds(r0, CH)], cbuf)
        copies = [
            pltpu.async_copy(
                table_hbm.at[cbuf.at[j]],
                rowsbuf.at[pl.ds(j * STREAM, STREAM)],
                sem,
            )
            for j in range(CH)
        ]
        for cp in copies:
            cp.wait()

        def tile(t, cc):
            for d in range(DIM):
                rbase = rowv[d] + 32 * t
                for blk in range(8):
                    v = plsc.load_gather(rowsbuf, [rbase + 4 * blk, colv[d]])
                    obuf[pl.ds(512 * t + 128 * d + 16 * blk, 16)] = v
            return cc

        lax.fori_loop(0, 32, tile, 0)
        pltpu.sync_copy(obuf, out_hbm.at[pl.ds(r0 * STREAM * GDIM, G_STEP * GDIM)])
        return carry

    lax.fori_loop(0, STEPS, step, 0)


def kernel(inputs, embed_weight):
    a = inputs.astype(jnp.int32).T
    table = _product_table(embed_weight.astype(jnp.float32))
    c = _combined_indices(a).reshape(C_ROWS, STREAM)
    mesh = plsc.VectorSubcoreMesh(
        core_axis_name="c", subcore_axis_name="s", num_cores=NC, num_subcores=NS
    )
    out1 = pl.kernel(
        _gather_body,
        out_type=jax.ShapeDtypeStruct((NOUT,), jnp.float32),
        mesh=mesh,
        scratch_types=[
            pltpu.VMEM((CH, STREAM), jnp.int32),
            pltpu.VMEM((G_STEP, GDIM), jnp.float32),
            pltpu.VMEM((G_STEP * GDIM,), jnp.float32),
            pltpu.SemaphoreType.DMA,
        ],
        compiler_params=pltpu.CompilerParams(
            use_tc_tiling_on_sc=False, needs_layout_passes=False
        ),
    )(table, c)
    out4 = out1.reshape(COLS, ROWS // 128, DIM, 128)
    return out4.transpose(1, 3, 0, 2).reshape(ROWS, COLS, DIM)


# speedup vs baseline: 82.1928x; 1.5716x over previous
import jax
import jax.numpy as jnp
from jax import lax
from jax.experimental import pallas as pl
from jax.experimental.pallas import tpu as pltpu
from jax.experimental.pallas import tpu_sc as plsc

ROWS, COLS = 16384, 200
VOCAB, DIM = 6, 4
PACK = 4
GDIM = PACK * DIM
NCOMB = VOCAB ** PACK
NG = ROWS * COLS // PACK
NOUT = ROWS * COLS * DIM

NC, NS = 2, 16
NW = NC * NS
STREAM = 128
CH = 8
G_STEP = CH * STREAM
G_W = NG // NW
STEPS = G_W // G_STEP
C_ROWS = NG // STREAM

_TCJ = 8
_KC = 512


def _cidx_body(a_ref, c_ref):
    kio = lax.broadcasted_iota(jnp.int32, (_KC, 128), 0)
    mio = lax.broadcasted_iota(jnp.int32, (_KC, 128), 1)
    a = kio % PACK
    wgt = jnp.where(a == 0, 216.0, jnp.where(a == 1, 36.0, jnp.where(a == 2, 6.0, 1.0)))
    m00 = jnp.where(kio // PACK == mio, wgt, 0.0)
    af = a_ref[...].astype(jnp.float32)
    for t in range(ROWS // _KC):
        blk = af[:, _KC * t:_KC * (t + 1)]
        c = jax.lax.dot(blk, m00, preferred_element_type=jnp.float32)
        c_ref[:, t, :] = c.astype(jnp.int32)


def _combined_indices(a):
    return pl.pallas_call(
        _cidx_body,
        out_shape=jax.ShapeDtypeStruct((COLS, ROWS // _KC, 128), jnp.int32),
        grid=(COLS // _TCJ,),
        in_specs=[pl.BlockSpec((_TCJ, ROWS), lambda i: (i, 0))],
        out_specs=pl.BlockSpec((_TCJ, ROWS // _KC, 128), lambda i: (i, 0, 0)),
    )(a)


def _product_table(embed_weight):
    ar = jnp.arange(NCOMB, dtype=jnp.int32)
    digs = jnp.stack(
        [ar // 216 % VOCAB, ar // 36 % VOCAB, ar // VOCAB % VOCAB, ar % VOCAB], axis=1
    )
    onehot = (digs[:, :, None] == jnp.arange(VOCAB)).astype(jnp.float32)
    table = jnp.sum(onehot[:, :, :, None] * embed_weight[None, None, :, :], axis=2)
    return table.reshape(NCOMB, GDIM)


def _gather_body(
    table_hbm, c_hbm, out_hbm, tspm, cbuf, rb0, rb1, ob0, ob1, sg0, sg1, so0, so1
):
    wid = lax.axis_index("s") * NC + lax.axis_index("c")

    @pl.when(lax.axis_index("s") == 0)
    def _():
        pltpu.sync_copy(table_hbm, tspm)

    plsc.subcore_barrier()
    pltpu.sync_copy(c_hbm.at[pl.ds(wid * G_W, G_W)], cbuf)

    iot4 = 4 * lax.iota(jnp.int32, 16)
    rbufs, obufs = (rb0, rb1), (ob0, ob1)
    gsems, osems = (sg0, sg1), (so0, so1)
    out_base = wid * G_W * GDIM

    def fire(s):
        b = s % 2
        return [
            pltpu.async_copy(
                tspm.at[cbuf.at[pl.ds(s * G_STEP + j * STREAM, STREAM)]],
                rbufs[b].at[pl.ds(j * STREAM, STREAM)],
                gsems[b],
            )
            for j in range(CH)
        ]

    gathers = {0: fire(0)}
    outcps = {}
    for s in range(STEPS):
        if s + 1 < STEPS:
            gathers[s + 1] = fire(s + 1)
        b = s % 2
        for cp in gathers.pop(s):
            cp.wait()
        if s - 2 in outcps:
            outcps.pop(s - 2).wait()
        rowsbuf, obuf = rbufs[b], obufs[b]

        def tile(q, cc):
            t, d = q >> 2, q & 3
            dv = d + iot4
            col = dv & 15
            rbase = (dv >> 4) + 32 * t
            dst = 512 * t + 128 * d
            for blk in range(8):
                v = plsc.load_gather(rowsbuf, [rbase + 4 * blk, col])
                obuf[pl.ds(dst + 16 * blk, 16)] = v
            return cc

        lax.fori_loop(0, 128, tile, 0)
        outcps[s] = pltpu.async_copy(
            obuf,
            out_hbm.at[pl.ds(out_base + s * G_STEP * GDIM, G_STEP * GDIM)],
            osems[b],
        )
    for cp in outcps.values():
        cp.wait()


def kernel(inputs, embed_weight):
    a = inputs.astype(jnp.int32).T
    table = _product_table(embed_weight.astype(jnp.float32))
    c = _combined_indices(a).reshape(NG)
    mesh = plsc.VectorSubcoreMesh(
        core_axis_name="c", subcore_axis_name="s", num_cores=NC, num_subcores=NS
    )
    out1 = pl.kernel(
        _gather_body,
        out_type=jax.ShapeDtypeStruct((NOUT,), jnp.float32),
        mesh=mesh,
        scratch_types=[
            pltpu.VMEM_SHARED((NCOMB, GDIM), jnp.float32),
            pltpu.VMEM((G_W,), jnp.int32),
            pltpu.VMEM((G_STEP, GDIM), jnp.float32),
            pltpu.VMEM((G_STEP, GDIM), jnp.float32),
            pltpu.VMEM((G_STEP * GDIM,), jnp.float32),
            pltpu.VMEM((G_STEP * GDIM,), jnp.float32),
            pltpu.SemaphoreType.DMA,
            pltpu.SemaphoreType.DMA,
            pltpu.SemaphoreType.DMA,
            pltpu.SemaphoreType.DMA,
        ],
        compiler_params=pltpu.CompilerParams(
            use_tc_tiling_on_sc=False, needs_layout_passes=False
        ),
    )(table, c)
    out4 = out1.reshape(COLS, ROWS // 128, DIM, 128)
    return out4.transpose(1, 3, 0, 2).reshape(ROWS, COLS, DIM)


# speedup vs baseline: 91.1346x; 1.1088x over previous
import jax
import jax.numpy as jnp
from jax import lax
from jax.experimental import pallas as pl
from jax.experimental.pallas import tpu as pltpu
from jax.experimental.pallas import tpu_sc as plsc

ROWS, COLS = 16384, 200
VOCAB, DIM = 6, 4
PACK = 4
GDIM = PACK * DIM
NCOMB = VOCAB ** PACK
NG = ROWS * COLS // PACK
NOUT = ROWS * COLS * DIM

NC, NS = 2, 16
NW = NC * NS
STREAM = 128
CH = 8
G_STEP = CH * STREAM
G_W = NG // NW
STEPS = G_W // G_STEP
C_ROWS = NG // STREAM

_TCJ = 40
_KC = 512


def _cidx_body(a_ref, c_ref):
    kio = lax.broadcasted_iota(jnp.int32, (_KC, 128), 0)
    mio = lax.broadcasted_iota(jnp.int32, (_KC, 128), 1)
    a = kio % PACK
    wgt = jnp.where(a == 0, 216.0, jnp.where(a == 1, 36.0, jnp.where(a == 2, 6.0, 1.0)))
    m00 = jnp.where(kio // PACK == mio, wgt, 0.0)
    af = a_ref[...].astype(jnp.float32)
    for t in range(ROWS // _KC):
        blk = af[:, _KC * t:_KC * (t + 1)]
        c = jax.lax.dot(blk, m00, preferred_element_type=jnp.float32)
        c_ref[:, t, :] = c.astype(jnp.int32)


def _combined_indices(a):
    return pl.pallas_call(
        _cidx_body,
        out_shape=jax.ShapeDtypeStruct((COLS, ROWS // _KC, 128), jnp.int32),
        grid=(COLS // _TCJ,),
        in_specs=[pl.BlockSpec((_TCJ, ROWS), lambda i: (i, 0))],
        out_specs=pl.BlockSpec((_TCJ, ROWS // _KC, 128), lambda i: (i, 0, 0)),
    )(a)


def _product_table(embed_weight):
    ar = jnp.arange(NCOMB, dtype=jnp.int32)
    digs = jnp.stack(
        [ar // 216 % VOCAB, ar // 36 % VOCAB, ar // VOCAB % VOCAB, ar % VOCAB], axis=1
    )
    onehot = (digs[:, :, None] == jnp.arange(VOCAB)).astype(jnp.float32)
    table = jnp.sum(onehot[:, :, :, None] * embed_weight[None, None, :, :], axis=2)
    return table.reshape(NCOMB, GDIM)


def _gather_body(
    table_hbm, c_hbm, out_hbm, tspm, cbuf, rb0, rb1, ob0, ob1, sg0, sg1, so0, so1
):
    wid = lax.axis_index("s") * NC + lax.axis_index("c")

    @pl.when(lax.axis_index("s") == 0)
    def _():
        pltpu.sync_copy(table_hbm, tspm)

    plsc.subcore_barrier()
    pltpu.sync_copy(c_hbm.at[pl.ds(wid * G_W, G_W)], cbuf)

    iot4 = 4 * lax.iota(jnp.int32, 16)
    rbufs, obufs = (rb0, rb1), (ob0, ob1)
    gsems, osems = (sg0, sg1), (so0, so1)
    out_base = wid * G_W * GDIM

    def fire(s):
        b = s % 2
        return [
            pltpu.async_copy(
                tspm.at[cbuf.at[pl.ds(s * G_STEP + j * STREAM, STREAM)]],
                rbufs[b].at[pl.ds(j * STREAM, STREAM)],
                gsems[b],
            )
            for j in range(CH)
        ]

    gathers = {0: fire(0)}
    outcps = {}
    for s in range(STEPS):
        if s + 1 < STEPS:
            gathers[s + 1] = fire(s + 1)
        b = s % 2
        for cp in gathers.pop(s):
            cp.wait()
        if s - 2 in outcps:
            outcps.pop(s - 2).wait()
        rowsbuf, obuf = rbufs[b], obufs[b]

        def tile(q, cc):
            t, d = q >> 2, q & 3
            dv = d + iot4
            col = dv & 15
            rbase = (dv >> 4) + 32 * t
            dst = 512 * t + 128 * d
            for blk in range(8):
                v = plsc.load_gather(rowsbuf, [rbase + 4 * blk, col])
                obuf[pl.ds(dst + 16 * blk, 16)] = v
            return cc

        lax.fori_loop(0, 128, tile, 0)
        outcps[s] = pltpu.async_copy(
            obuf,
            out_hbm.at[pl.ds(out_base + s * G_STEP * GDIM, G_STEP * GDIM)],
            osems[b],
        )
    for cp in outcps.values():
        cp.wait()


def kernel(inputs, embed_weight):
    a = inputs.astype(jnp.int32).T
    table = _product_table(embed_weight.astype(jnp.float32))
    c = _combined_indices(a).reshape(NG)
    mesh = plsc.VectorSubcoreMesh(
        core_axis_name="c", subcore_axis_name="s", num_cores=NC, num_subcores=NS
    )
    out1 = pl.kernel(
        _gather_body,
        out_type=jax.ShapeDtypeStruct((NOUT,), jnp.float32),
        mesh=mesh,
        scratch_types=[
            pltpu.VMEM_SHARED((NCOMB, GDIM), jnp.float32),
            pltpu.VMEM((G_W,), jnp.int32),
            pltpu.VMEM((G_STEP, GDIM), jnp.float32),
            pltpu.VMEM((G_STEP, GDIM), jnp.float32),
            pltpu.VMEM((G_STEP * GDIM,), jnp.float32),
            pltpu.VMEM((G_STEP * GDIM,), jnp.float32),
            pltpu.SemaphoreType.DMA,
            pltpu.SemaphoreType.DMA,
            pltpu.SemaphoreType.DMA,
            pltpu.SemaphoreType.DMA,
        ],
        compiler_params=pltpu.CompilerParams(
            use_tc_tiling_on_sc=False, needs_layout_passes=False
        ),
    )(table, c)
    out4 = out1.reshape(COLS, ROWS // 128, DIM, 128)
    return out4.transpose(1, 3, 0, 2).reshape(ROWS, COLS, DIM)
